# baseline (speedup 1.0000x reference)
import functools

import jax
import jax.numpy as jnp
from jax.experimental import pallas as pl
from jax.experimental.pallas import tpu as pltpu
from jax.experimental.pallas import tpu_sc as plsc

E = 8
K = 2
D = 1024
F = 2048
T = 4096
A = K * T

B_BLK = 256
S = A + E * B_BLK
NB = S // B_BLK
F_BLK = 2048
NF = F // F_BLK

NW = 32
T_PER_W = T // NW
CH_D = 64
CH_C = 32

_FP32 = jnp.float32
_I32 = jnp.int32


def _router_body(x_ref, wg_ref, pos_ref, wts_ref, be_ref):
    x = x_ref[...]
    wg = wg_ref[...]
    lt = jax.lax.dot_general(
        wg, x, (((1,), (1,)), ((), ())), preferred_element_type=_FP32)

    row = jax.lax.broadcasted_iota(_I32, (E, T), 0)
    m1 = jnp.max(lt, axis=0, keepdims=True)
    e1 = jnp.min(jnp.where(lt == m1, row, E), axis=0, keepdims=True)
    masked = jnp.where(row == e1, -jnp.inf, lt)
    m2 = jnp.max(masked, axis=0, keepdims=True)
    e2 = jnp.min(jnp.where(masked == m2, row, E), axis=0, keepdims=True)

    t = jnp.exp(m2 - m1)
    w_hi = 1.0 / (1.0 + t)
    w_lo = 1.0 - w_hi

    R, L = A // 128, 128
    e_lay = jnp.concatenate([e1, e2], axis=0).reshape(R, L)
    wts_ref[...] = jnp.concatenate([w_hi, w_lo], axis=0).reshape(R, L)

    li = jax.lax.broadcasted_iota(_I32, (L, L), 0)
    lj = jax.lax.broadcasted_iota(_I32, (L, L), 1)
    U = (li <= lj).astype(_FP32)
    ri = jax.lax.broadcasted_iota(_I32, (R, R), 0)
    rj = jax.lax.broadcasted_iota(_I32, (R, R), 1)
    Lo = (rj < ri).astype(_FP32)

    blk = jax.lax.broadcasted_iota(_I32, (1, 128), 1).astype(_FP32) * B_BLK
    pos_acc = jnp.zeros((R, L), _FP32)
    be_acc = jnp.zeros((1, 128), _I32)
    start = jnp.zeros((1, 1), _FP32)
    for e in range(E):
        m = (e_lay == e).astype(_FP32)
        p = jax.lax.dot_general(
            m, U, (((1,), (0,)), ((), ())), preferred_element_type=_FP32)
        tot = p[:, L - 1 : L]
        carry = jax.lax.dot_general(
            Lo, tot, (((1,), (0,)), ((), ())), preferred_element_type=_FP32)
        count = carry[R - 1 : R, :] + tot[R - 1 : R, :]
        pos_e = p - 1.0 + carry + start
        pos_acc = pos_acc + m * pos_e
        if e >= 1:
            be_acc = be_acc + (blk >= start).astype(_I32)
        start = start + jnp.ceil(count / B_BLK) * B_BLK
    pos_ref[...] = pos_acc.astype(_I32)

    lane = jax.lax.broadcasted_iota(_I32, (1, 128), 1)
    in_nb = lane < NB
    prev = jnp.concatenate([jnp.full((1, 1), -1, _I32), be_acc[:, :127]], axis=1)
    seg_first = jnp.where(in_nb, (be_acc != prev).astype(_I32), 0)
    sidx = seg_first
    s = 1
    while s < 128:
        sidx = sidx + jnp.concatenate(
            [jnp.zeros((1, s), _I32), sidx[:, : 128 - s]], axis=1)
        s *= 2
    sidx = sidx - 1
    nseg = jnp.sum(seg_first, axis=1, keepdims=True)
    seg_e = jnp.zeros((1, 128), _I32)
    seg_end = jnp.zeros((1, 128), _I32)
    for k in range(E):
        is_k = (sidx == k) & (seg_first == 1)
        e_k = jnp.sum(jnp.where(is_k, be_acc, 0), axis=1, keepdims=True)
        end_k = jnp.sum(
            jnp.where((sidx <= k) & in_nb, 1, 0), axis=1, keepdims=True)
        seg_e = jnp.where(lane == k, e_k, seg_e)
        seg_end = jnp.where(lane == k, end_k, seg_end)
    be_ref[...] = jnp.concatenate(
        [be_acc, seg_e, seg_end, jnp.broadcast_to(nseg, (1, 128)),
         jnp.zeros((4, 128), _I32)], axis=0)


def _router(inputs, Wg):
    return pl.pallas_call(
        _router_body,
        out_shape=[
            jax.ShapeDtypeStruct((A // 128, 128), _I32),
            jax.ShapeDtypeStruct((A // 128, 128), _FP32),
            jax.ShapeDtypeStruct((8, 128), _I32),
        ],
    )(inputs, Wg)


def _dispatch_body(x_hbm, pos_hbm, xd_hbm, idx_v, rows_v):
    wid = jax.lax.axis_index("s") * 2 + jax.lax.axis_index("c")

    @pl.loop(0, T_PER_W // CH_D)
    def _(it):
        base = wid * T_PER_W + it * CH_D
        pltpu.sync_copy(x_hbm.at[pl.ds(base, CH_D)], rows_v)
        for j in range(K):
            pltpu.sync_copy(pos_hbm.at[pl.ds(j * T + base, CH_D)], idx_v)
            pltpu.sync_copy(rows_v, xd_hbm.at[idx_v])


@functools.lru_cache(maxsize=1)
def _sc_kernels():
    mesh = plsc.VectorSubcoreMesh(core_axis_name="c", subcore_axis_name="s")
    dispatch = functools.partial(
        pl.kernel,
        mesh=mesh,
        out_type=jax.ShapeDtypeStruct((S, D), _FP32),
        scratch_types=[
            pltpu.VMEM((CH_D,), _I32),
            pltpu.VMEM((CH_D, D), _FP32),
        ],
    )(_dispatch_body)
    combine = functools.partial(
        pl.kernel,
        mesh=mesh,
        out_type=jax.ShapeDtypeStruct((T, D), _FP32),
        compiler_params=pltpu.CompilerParams(needs_layout_passes=False),
        scratch_types=[
            pltpu.VMEM((CH_C,), _I32),
            pltpu.VMEM((CH_C,), _I32),
            pltpu.VMEM((CH_C,), _FP32),
            pltpu.VMEM((CH_C,), _FP32),
            pltpu.VMEM((CH_C, D), _FP32),
            pltpu.VMEM((CH_C, D), _FP32),
            pltpu.VMEM((CH_C, D), _FP32),
            pltpu.SemaphoreType.DMA,
            pltpu.SemaphoreType.DMA,
        ],
    )(_combine_body)
    return dispatch, combine


def _ffn_body(meta_ref, x_hbm, w1_hbm, w3_hbm, w2_hbm, y_hbm,
              xb, yb, w1s, w3s, w2s, sem_x, sem_w, sem_y):
    nseg = meta_ref[3, 0]

    def fetch_w(e, slot):
        pltpu.make_async_copy(w1_hbm.at[e], w1s.at[slot], sem_w.at[slot]).start()
        pltpu.make_async_copy(w3_hbm.at[e], w3s.at[slot], sem_w.at[slot]).start()
        pltpu.make_async_copy(w2_hbm.at[e], w2s.at[slot], sem_w.at[slot]).start()

    def wait_w(e, slot):
        pltpu.make_async_copy(w1_hbm.at[e], w1s.at[slot], sem_w.at[slot]).wait()
        pltpu.make_async_copy(w3_hbm.at[e], w3s.at[slot], sem_w.at[slot]).wait()
        pltpu.make_async_copy(w2_hbm.at[e], w2s.at[slot], sem_w.at[slot]).wait()

    def start_x(b):
        s = jax.lax.rem(b, 2)
        pltpu.make_async_copy(
            x_hbm.at[pl.ds(b * B_BLK, B_BLK)], xb.at[s], sem_x.at[s]).start()

    fetch_w(meta_ref[1, 0], 0)
    start_x(0)

    for k in range(E):
        w_slot = k % 2

        @pl.when(k < nseg)
        def _(k=k, w_slot=w_slot):
            wait_w(meta_ref[1, k], w_slot)

            @pl.when(k + 1 < nseg)
            def _():
                fetch_w(meta_ref[1, k + 1], 1 - w_slot)

        def blk(b, carry, w_slot=w_slot):
            s = jax.lax.rem(b, 2)

            @pl.when(b + 1 < NB)
            def _():
                start_x(b + 1)

            pltpu.make_async_copy(
                x_hbm.at[pl.ds(b * B_BLK, B_BLK)], xb.at[s], sem_x.at[s]).wait()
            x = xb[s]
            a = jax.lax.dot_general(
                x, w1s[w_slot], (((1,), (1,)), ((), ())),
                preferred_element_type=_FP32)
            c = jax.lax.dot_general(
                x, w3s[w_slot], (((1,), (1,)), ((), ())),
                preferred_element_type=_FP32)
            h = (a * (0.5 * jnp.tanh(0.5 * a) + 0.5)) * c
            y = jax.lax.dot_general(
                h, w2s[w_slot], (((1,), (1,)), ((), ())),
                preferred_element_type=_FP32)

            @pl.when(b >= 2)
            def _():
                pltpu.make_async_copy(
                    yb.at[s], y_hbm.at[pl.ds((b - 2) * B_BLK, B_BLK)],
                    sem_y.at[s]).wait()

            yb[s] = y
            pltpu.make_async_copy(
                yb.at[s], y_hbm.at[pl.ds(b * B_BLK, B_BLK)], sem_y.at[s]).start()
            return carry

        lo = jnp.int32(0) if k == 0 else meta_ref[2, k - 1]
        jax.lax.fori_loop(lo, meta_ref[2, k], blk, 0)

    for b in (NB - 2, NB - 1):
        s = b % 2
        pltpu.make_async_copy(
            yb.at[s], y_hbm.at[pl.ds(b * B_BLK, B_BLK)], sem_y.at[s]).wait()


def _ffn(meta, xd, w1, w2, w3):
    return pl.pallas_call(
        _ffn_body,
        in_specs=[
            pl.BlockSpec(memory_space=pltpu.SMEM),
            pl.BlockSpec(memory_space=pl.ANY),
            pl.BlockSpec(memory_space=pl.ANY),
            pl.BlockSpec(memory_space=pl.ANY),
            pl.BlockSpec(memory_space=pl.ANY),
        ],
        out_specs=pl.BlockSpec(memory_space=pl.ANY),
        out_shape=jax.ShapeDtypeStruct((S, D), _FP32),
        scratch_shapes=[
            pltpu.VMEM((2, B_BLK, D), _FP32),
            pltpu.VMEM((2, B_BLK, D), _FP32),
            pltpu.VMEM((2, F_BLK, D), _FP32),
            pltpu.VMEM((2, F_BLK, D), _FP32),
            pltpu.VMEM((2, D, F_BLK), _FP32),
            pltpu.SemaphoreType.DMA((2,)),
            pltpu.SemaphoreType.DMA((2,)),
            pltpu.SemaphoreType.DMA((2,)),
        ],
    )(meta, xd, w1, w3, w2)


def _combine_body(y_hbm, pos_hbm, w_hbm, out_hbm,
                  idx0, idx1, w0v, w1v, g0, g1, ov, sem0, sem1):
    wid = jax.lax.axis_index("s") * 2 + jax.lax.axis_index("c")

    @pl.loop(0, T_PER_W // CH_C)
    def _(it):
        base = wid * T_PER_W + it * CH_C
        pltpu.sync_copy(pos_hbm.at[pl.ds(base, CH_C)], idx0)
        pltpu.sync_copy(pos_hbm.at[pl.ds(T + base, CH_C)], idx1)
        pltpu.sync_copy(w_hbm.at[pl.ds(base, CH_C)], w0v)
        pltpu.sync_copy(w_hbm.at[pl.ds(T + base, CH_C)], w1v)
        cp0 = pltpu.async_copy(y_hbm.at[idx0], g0, sem0)
        cp1 = pltpu.async_copy(y_hbm.at[idx1], g1, sem1)
        cp0.wait()
        cp1.wait()

        @pl.loop(0, CH_C)
        def _(r):
            lane = jnp.full((16,), r, _I32)
            w0s = plsc.load_gather(w0v, [lane])
            w1s = plsc.load_gather(w1v, [lane])

            @pl.loop(0, D, step=16)
            def _(c):
                ov[r, pl.ds(c, 16)] = (
                    g0[r, pl.ds(c, 16)] * w0s + g1[r, pl.ds(c, 16)] * w1s)

        pltpu.sync_copy(ov, out_hbm.at[pl.ds(base, CH_C)])


def kernel(inputs, Wg, w1, w2, w3):
    pos, wts, meta = _router(inputs, Wg)
    pos1 = pos.reshape(A)
    wts1 = wts.reshape(A)
    dispatch, combine = _sc_kernels()
    xd = dispatch(inputs, pos1)
    yd = _ffn(meta, xd, w1, w2, w3)
    return combine(yd, pos1, wts1)


# speedup vs baseline: 1.0631x; 1.0631x over previous
import functools

import jax
import jax.numpy as jnp
from jax.experimental import pallas as pl
from jax.experimental.pallas import tpu as pltpu
from jax.experimental.pallas import tpu_sc as plsc

E = 8
K = 2
D = 1024
F = 2048
T = 4096
A = K * T

B_BLK = 256
S = A + E * B_BLK
NB = S // B_BLK
F_BLK = 2048
NF = F // F_BLK

NW = 32
T_PER_W = T // NW
CH_D = 64
CH_C = 32

_FP32 = jnp.float32
_I32 = jnp.int32


def _router_body(x_ref, wg_ref, pos_ref, wts_ref, be_ref):
    x = x_ref[...]
    wg = wg_ref[...]
    lt = jax.lax.dot_general(
        wg, x, (((1,), (1,)), ((), ())), preferred_element_type=_FP32)

    row = jax.lax.broadcasted_iota(_I32, (E, T), 0)
    m1 = jnp.max(lt, axis=0, keepdims=True)
    e1 = jnp.min(jnp.where(lt == m1, row, E), axis=0, keepdims=True)
    masked = jnp.where(row == e1, -jnp.inf, lt)
    m2 = jnp.max(masked, axis=0, keepdims=True)
    e2 = jnp.min(jnp.where(masked == m2, row, E), axis=0, keepdims=True)

    t = jnp.exp(m2 - m1)
    w_hi = 1.0 / (1.0 + t)
    w_lo = 1.0 - w_hi

    R, L = A // 128, 128
    e_lay = jnp.concatenate([e1, e2], axis=0).reshape(R, L)
    wts_ref[...] = jnp.concatenate([w_hi, w_lo], axis=0).reshape(R, L)

    li = jax.lax.broadcasted_iota(_I32, (L, L), 0)
    lj = jax.lax.broadcasted_iota(_I32, (L, L), 1)
    U = (li <= lj).astype(_FP32)
    ri = jax.lax.broadcasted_iota(_I32, (R, R), 0)
    rj = jax.lax.broadcasted_iota(_I32, (R, R), 1)
    Lo = (rj < ri).astype(_FP32)

    blk = jax.lax.broadcasted_iota(_I32, (1, 128), 1).astype(_FP32) * B_BLK
    pos_acc = jnp.zeros((R, L), _FP32)
    be_acc = jnp.zeros((1, 128), _I32)
    start = jnp.zeros((1, 1), _FP32)
    for e in range(E):
        m = (e_lay == e).astype(_FP32)
        p = jax.lax.dot_general(
            m, U, (((1,), (0,)), ((), ())), preferred_element_type=_FP32)
        tot = p[:, L - 1 : L]
        carry = jax.lax.dot_general(
            Lo, tot, (((1,), (0,)), ((), ())), preferred_element_type=_FP32)
        count = carry[R - 1 : R, :] + tot[R - 1 : R, :]
        pos_e = p - 1.0 + carry + start
        pos_acc = pos_acc + m * pos_e
        if e >= 1:
            be_acc = be_acc + (blk >= start).astype(_I32)
        start = start + jnp.ceil(count / B_BLK) * B_BLK
    pos_ref[...] = pos_acc.astype(_I32)

    lane = jax.lax.broadcasted_iota(_I32, (1, 128), 1)
    in_nb = (lane < NB) & (blk < start)
    prev = jnp.concatenate([jnp.full((1, 1), -1, _I32), be_acc[:, :127]], axis=1)
    seg_first = jnp.where(in_nb, (be_acc != prev).astype(_I32), 0)
    sidx = seg_first
    s = 1
    while s < 128:
        sidx = sidx + jnp.concatenate(
            [jnp.zeros((1, s), _I32), sidx[:, : 128 - s]], axis=1)
        s *= 2
    sidx = sidx - 1
    nseg = jnp.sum(seg_first, axis=1, keepdims=True)
    seg_e = jnp.zeros((1, 128), _I32)
    seg_end = jnp.zeros((1, 128), _I32)
    for k in range(E):
        is_k = (sidx == k) & (seg_first == 1)
        e_k = jnp.sum(jnp.where(is_k, be_acc, 0), axis=1, keepdims=True)
        end_k = jnp.sum(
            jnp.where((sidx <= k) & in_nb, 1, 0), axis=1, keepdims=True)
        seg_e = jnp.where(lane == k, e_k, seg_e)
        seg_end = jnp.where(lane == k, end_k, seg_end)
    be_ref[...] = jnp.concatenate(
        [be_acc, seg_e, seg_end, jnp.broadcast_to(nseg, (1, 128)),
         jnp.zeros((4, 128), _I32)], axis=0)


def _router(inputs, Wg):
    return pl.pallas_call(
        _router_body,
        out_shape=[
            jax.ShapeDtypeStruct((A // 128, 128), _I32),
            jax.ShapeDtypeStruct((A // 128, 128), _FP32),
            jax.ShapeDtypeStruct((8, 128), _I32),
        ],
    )(inputs, Wg)


def _dispatch_body(x_hbm, pos_hbm, xd_hbm, idx_v, rows_v):
    wid = jax.lax.axis_index("s") * 2 + jax.lax.axis_index("c")

    @pl.loop(0, T_PER_W // CH_D)
    def _(it):
        base = wid * T_PER_W + it * CH_D
        pltpu.sync_copy(x_hbm.at[pl.ds(base, CH_D)], rows_v)
        for j in range(K):
            pltpu.sync_copy(pos_hbm.at[pl.ds(j * T + base, CH_D)], idx_v)
            pltpu.sync_copy(rows_v, xd_hbm.at[idx_v])


@functools.lru_cache(maxsize=1)
def _sc_kernels():
    mesh = plsc.VectorSubcoreMesh(core_axis_name="c", subcore_axis_name="s")
    dispatch = functools.partial(
        pl.kernel,
        mesh=mesh,
        out_type=jax.ShapeDtypeStruct((S, D), _FP32),
        scratch_types=[
            pltpu.VMEM((CH_D,), _I32),
            pltpu.VMEM((CH_D, D), _FP32),
        ],
    )(_dispatch_body)
    combine = functools.partial(
        pl.kernel,
        mesh=mesh,
        out_type=jax.ShapeDtypeStruct((T, D), _FP32),
        compiler_params=pltpu.CompilerParams(needs_layout_passes=False),
        scratch_types=[
            pltpu.VMEM((CH_C,), _I32),
            pltpu.VMEM((CH_C,), _I32),
            pltpu.VMEM((CH_C,), _FP32),
            pltpu.VMEM((CH_C,), _FP32),
            pltpu.VMEM((CH_C, D), _FP32),
            pltpu.VMEM((CH_C, D), _FP32),
            pltpu.VMEM((CH_C, D), _FP32),
            pltpu.SemaphoreType.DMA,
            pltpu.SemaphoreType.DMA,
        ],
    )(_combine_body)
    return dispatch, combine


def _ffn_body(meta_ref, x_hbm, w1_hbm, w3_hbm, w2_hbm, y_hbm,
              xb, yb, w1s, w3s, w2s, sem_x, sem_w, sem_y):
    nseg = meta_ref[3, 0]
    nblk = meta_ref[2, nseg - 1]

    def fetch_w(e, slot):
        pltpu.make_async_copy(w1_hbm.at[e], w1s.at[slot], sem_w.at[slot]).start()
        pltpu.make_async_copy(w3_hbm.at[e], w3s.at[slot], sem_w.at[slot]).start()
        pltpu.make_async_copy(w2_hbm.at[e], w2s.at[slot], sem_w.at[slot]).start()

    def wait_w(e, slot):
        pltpu.make_async_copy(w1_hbm.at[e], w1s.at[slot], sem_w.at[slot]).wait()
        pltpu.make_async_copy(w3_hbm.at[e], w3s.at[slot], sem_w.at[slot]).wait()
        pltpu.make_async_copy(w2_hbm.at[e], w2s.at[slot], sem_w.at[slot]).wait()

    def start_x(b):
        s = jax.lax.rem(b, 2)
        pltpu.make_async_copy(
            x_hbm.at[pl.ds(b * B_BLK, B_BLK)], xb.at[s], sem_x.at[s]).start()

    fetch_w(meta_ref[1, 0], 0)
    start_x(0)

    for k in range(E):
        w_slot = k % 2

        @pl.when(k < nseg)
        def _(k=k, w_slot=w_slot):
            wait_w(meta_ref[1, k], w_slot)

            @pl.when(k + 1 < nseg)
            def _():
                fetch_w(meta_ref[1, k + 1], 1 - w_slot)

        def blk(b, carry, w_slot=w_slot):
            s = jax.lax.rem(b, 2)

            @pl.when(b + 1 < nblk)
            def _():
                start_x(b + 1)

            pltpu.make_async_copy(
                x_hbm.at[pl.ds(b * B_BLK, B_BLK)], xb.at[s], sem_x.at[s]).wait()
            x = xb[s]
            a = jax.lax.dot_general(
                x, w1s[w_slot], (((1,), (1,)), ((), ())),
                preferred_element_type=_FP32)
            c = jax.lax.dot_general(
                x, w3s[w_slot], (((1,), (1,)), ((), ())),
                preferred_element_type=_FP32)
            h = (a * (0.5 * jnp.tanh(0.5 * a) + 0.5)) * c
            y = jax.lax.dot_general(
                h, w2s[w_slot], (((1,), (1,)), ((), ())),
                preferred_element_type=_FP32)

            @pl.when(b >= 2)
            def _():
                pltpu.make_async_copy(
                    yb.at[s], y_hbm.at[pl.ds((b - 2) * B_BLK, B_BLK)],
                    sem_y.at[s]).wait()

            yb[s] = y
            pltpu.make_async_copy(
                yb.at[s], y_hbm.at[pl.ds(b * B_BLK, B_BLK)], sem_y.at[s]).start()
            return carry

        lo = jnp.int32(0) if k == 0 else meta_ref[2, k - 1]
        jax.lax.fori_loop(lo, meta_ref[2, k], blk, 0)

    for i in (2, 1):
        b = nblk - i
        s = jax.lax.rem(b, 2)
        pltpu.make_async_copy(
            yb.at[s], y_hbm.at[pl.ds(b * B_BLK, B_BLK)], sem_y.at[s]).wait()


def _ffn(meta, xd, w1, w2, w3):
    return pl.pallas_call(
        _ffn_body,
        in_specs=[
            pl.BlockSpec(memory_space=pltpu.SMEM),
            pl.BlockSpec(memory_space=pl.ANY),
            pl.BlockSpec(memory_space=pl.ANY),
            pl.BlockSpec(memory_space=pl.ANY),
            pl.BlockSpec(memory_space=pl.ANY),
        ],
        out_specs=pl.BlockSpec(memory_space=pl.ANY),
        out_shape=jax.ShapeDtypeStruct((S, D), _FP32),
        scratch_shapes=[
            pltpu.VMEM((2, B_BLK, D), _FP32),
            pltpu.VMEM((2, B_BLK, D), _FP32),
            pltpu.VMEM((2, F_BLK, D), _FP32),
            pltpu.VMEM((2, F_BLK, D), _FP32),
            pltpu.VMEM((2, D, F_BLK), _FP32),
            pltpu.SemaphoreType.DMA((2,)),
            pltpu.SemaphoreType.DMA((2,)),
            pltpu.SemaphoreType.DMA((2,)),
        ],
    )(meta, xd, w1, w3, w2)


def _combine_body(y_hbm, pos_hbm, w_hbm, out_hbm,
                  idx0, idx1, w0v, w1v, g0, g1, ov, sem0, sem1):
    wid = jax.lax.axis_index("s") * 2 + jax.lax.axis_index("c")

    @pl.loop(0, T_PER_W // CH_C)
    def _(it):
        base = wid * T_PER_W + it * CH_C
        pltpu.sync_copy(pos_hbm.at[pl.ds(base, CH_C)], idx0)
        pltpu.sync_copy(pos_hbm.at[pl.ds(T + base, CH_C)], idx1)
        pltpu.sync_copy(w_hbm.at[pl.ds(base, CH_C)], w0v)
        pltpu.sync_copy(w_hbm.at[pl.ds(T + base, CH_C)], w1v)
        cp0 = pltpu.async_copy(y_hbm.at[idx0], g0, sem0)
        cp1 = pltpu.async_copy(y_hbm.at[idx1], g1, sem1)
        cp0.wait()
        cp1.wait()

        @pl.loop(0, CH_C)
        def _(r):
            lane = jnp.full((16,), r, _I32)
            w0s = plsc.load_gather(w0v, [lane])
            w1s = plsc.load_gather(w1v, [lane])

            @pl.loop(0, D, step=16)
            def _(c):
                ov[r, pl.ds(c, 16)] = (
                    g0[r, pl.ds(c, 16)] * w0s + g1[r, pl.ds(c, 16)] * w1s)

        pltpu.sync_copy(ov, out_hbm.at[pl.ds(base, CH_C)])


def kernel(inputs, Wg, w1, w2, w3):
    pos, wts, meta = _router(inputs, Wg)
    pos1 = pos.reshape(A)
    wts1 = wts.reshape(A)
    dispatch, combine = _sc_kernels()
    xd = dispatch(inputs, pos1)
    yd = _ffn(meta, xd, w1, w2, w3)
    return combine(yd, pos1, wts1)
